# Initial kernel scaffold; baseline (speedup 1.0000x reference)
import jax
import jax.numpy as jnp
from jax.experimental import pallas as pl


def kernel(x, W, b):
    raise NotImplementedError("write your pallas kernel here")


# speedup vs baseline: 2.9895x; 2.9895x over previous
import functools

import jax
import jax.numpy as jnp
from jax.experimental import pallas as pl


def _router_block(x_ref, w_ref, b_ref, out_ref):
    logits = jnp.dot(x_ref[...], w_ref[...], preferred_element_type=jnp.float32)
    logits = logits + b_ref[...]

    t, e = logits.shape
    idx = jax.lax.broadcasted_iota(jnp.int32, (t, e), 1)

    m1 = jnp.max(logits, axis=1, keepdims=True)
    is1 = logits == m1
    i1 = jnp.min(jnp.where(is1, idx, e), axis=1, keepdims=True)
    masked = jnp.where(idx == i1, -jnp.inf, logits)
    m2 = jnp.max(masked, axis=1, keepdims=True)
    is2 = masked == m2
    i2 = jnp.min(jnp.where(is2, idx, e), axis=1, keepdims=True)

    lse = m1 + jnp.log(jnp.sum(jnp.exp(logits - m1), axis=1, keepdims=True))
    p1 = jnp.exp(m1 - lse)
    p2 = jnp.exp(m2 - lse)

    g2 = 1.0 / (1.0 + jnp.exp(p1 - p2))
    g1 = 1.0 - g2

    out = jnp.where(idx == i1, g1, jnp.where(idx == i2, g2, 0.0))
    out_ref[...] = out


@jax.jit
def kernel(x, W, b):
    tokens, d_model = x.shape
    n_experts = W.shape[0]
    tile = 1024
    grid = (tokens // tile,)
    return pl.pallas_call(
        _router_block,
        grid=grid,
        in_specs=[
            pl.BlockSpec((tile, d_model), lambda i: (i, 0)),
            pl.BlockSpec((d_model, n_experts), lambda i: (0, 0)),
            pl.BlockSpec((1, n_experts), lambda i: (0, 0)),
        ],
        out_specs=pl.BlockSpec((tile, n_experts), lambda i: (i, 0)),
        out_shape=jax.ShapeDtypeStruct((tokens, n_experts), jnp.float32),
    )(x, W.T, b[None, :])
